# baseline (speedup 1.0000x reference)
import jax
import jax.numpy as jnp
from jax.experimental import pallas as pl
from jax.experimental.pallas import tpu as pltpu

_CHUNK = 1000


def _mlp_body(x_hbm, w1_ref, b1_ref, w2_ref, b2_ref, o_hbm):
    n_chunks = x_hbm.shape[0] // _CHUNK

    def inner(x_blk, o_blk):
        h = jnp.dot(x_blk[:].astype(jnp.bfloat16),
                    w1_ref[:].astype(jnp.bfloat16),
                    preferred_element_type=jnp.float32)
        h = h + b1_ref[:]
        h = jnp.where(h > 0, h, jnp.exp(h) - 1.0)
        o = jnp.dot(h.astype(jnp.bfloat16),
                    w2_ref[:].astype(jnp.bfloat16),
                    preferred_element_type=jnp.float32)
        o_blk[:] = o + b2_ref[:]

    pipeline = pltpu.emit_pipeline(
        inner,
        grid=(n_chunks,),
        in_specs=[pl.BlockSpec((_CHUNK, x_hbm.shape[1]), lambda i: (i, 0))],
        out_specs=[pl.BlockSpec((_CHUNK, x_hbm.shape[1]), lambda i: (i, 0))],
    )
    pipeline(x_hbm, o_hbm)


def kernel(x, edge_index, W1, b1, W2, b2):
    del edge_index
    N, D = x.shape
    return pl.pallas_call(
        _mlp_body,
        in_specs=[
            pl.BlockSpec(memory_space=pltpu.MemorySpace.HBM),
            pl.BlockSpec(memory_space=pltpu.MemorySpace.VMEM),
            pl.BlockSpec(memory_space=pltpu.MemorySpace.VMEM),
            pl.BlockSpec(memory_space=pltpu.MemorySpace.VMEM),
            pl.BlockSpec(memory_space=pltpu.MemorySpace.VMEM),
        ],
        out_specs=pl.BlockSpec(memory_space=pltpu.MemorySpace.HBM),
        out_shape=jax.ShapeDtypeStruct((N, D), x.dtype),
    )(x, W1, b1.reshape(1, D), W2, b2.reshape(1, D))
